# Initial kernel scaffold; baseline (speedup 1.0000x reference)
import jax
import jax.numpy as jnp
from jax.experimental import pallas as pl


def kernel(input, weights):
    raise NotImplementedError("write your pallas kernel here")


# speedup vs baseline: 2.2736x; 2.2736x over previous
import functools

import jax
import jax.numpy as jnp
from jax import lax
from jax.experimental import pallas as pl
from jax.experimental.pallas import tpu as pltpu
from jax.experimental.pallas import tpu_sc as plsc

PAD = 1
LANES = 16
NUM_CORES = 2
NUM_SUBCORES = 16
NUM_WORKERS = NUM_CORES * NUM_SUBCORES


def _build(bsz, seq_len, dim):
  n_tok = bsz * seq_len
  per_w = n_tok // NUM_WORKERS
  n_vregs = per_w // LANES
  G = 32
  n_chunks = per_w // G
  w_per_row = seq_len // per_w

  mesh = plsc.VectorSubcoreMesh(
      core_axis_name="c", subcore_axis_name="s", num_cores=NUM_CORES,
      num_subcores=NUM_SUBCORES)

  @functools.partial(
      pl.kernel,
      mesh=mesh,
      compiler_params=pltpu.CompilerParams(needs_layout_passes=False),
      out_type=jax.ShapeDtypeStruct((n_tok, dim), jnp.float32),
      scratch_types=[
          pltpu.VMEM((per_w,), jnp.int32),
          pltpu.VMEM((per_w,), jnp.int32),
          pltpu.VMEM((G, dim), jnp.float32),
          pltpu.VMEM((G, dim), jnp.float32),
          pltpu.VMEM((LANES,), jnp.int32),
          pltpu.VMEM((NUM_SUBCORES * LANES,), jnp.int32),
          pltpu.VMEM_SHARED((NUM_SUBCORES * LANES,), jnp.int32),
          pltpu.SemaphoreType.DMA,
          pltpu.SemaphoreType.DMA,
          pltpu.SemaphoreType.DMA,
          pltpu.SemaphoreType.DMA,
      ],
  )
  def k(ids_hbm, table_hbm, out_hbm, ids_v, pos_v, buf0, buf1, stage_v,
        tot_v, tot_sh, gsem0, gsem1, osem0, osem1):
    cid = lax.axis_index("c")
    sid = lax.axis_index("s")
    wid = cid * NUM_SUBCORES + sid
    base = wid * per_w

    pltpu.sync_copy(ids_hbm.at[pl.ds(base, per_w)], ids_v)

    def body(i, carry):
      ids = ids_v[pl.ds(i * LANES, LANES)]
      m = jnp.minimum(jnp.abs(ids - PAD), 1)
      c = plsc.cumsum(m)
      pos_v[pl.ds(i * LANES, LANES)] = c + carry
      return carry + jnp.sum(m)

    total = lax.fori_loop(0, n_vregs, body, jnp.int32(0))

    stage_v[...] = jnp.full((LANES,), total, jnp.int32)
    pltpu.sync_copy(stage_v, tot_sh.at[pl.ds(sid * LANES, LANES)])
    plsc.subcore_barrier()

    pltpu.sync_copy(tot_sh, tot_v)
    r0 = (sid // w_per_row) * w_per_row
    offset = lax.fori_loop(
        r0, sid, lambda j, acc: acc + tot_v[pl.ds(j * LANES, LANES)],
        jnp.zeros((LANES,), jnp.int32))

    def body2(i, carry):
      ids = ids_v[pl.ds(i * LANES, LANES)]
      m = jnp.minimum(jnp.abs(ids - PAD), 1)
      c = pos_v[pl.ds(i * LANES, LANES)]
      pos_v[pl.ds(i * LANES, LANES)] = (c + offset) * m + PAD
      return carry

    lax.fori_loop(0, n_vregs, body2, 0)

    bufs = (buf0, buf1)
    gsems = (gsem0, gsem1)
    osems = (osem0, osem1)
    gh = [None, None]
    oh = [None, None]
    for g in range(n_chunks):
      b = g & 1
      if oh[b] is not None:
        oh[b].wait()
      gh[b] = pltpu.async_copy(
          table_hbm.at[pos_v.at[pl.ds(g * G, G)]], bufs[b], gsems[b])
      pb = (g - 1) & 1
      if g >= 1:
        gh[pb].wait()
        oh[pb] = pltpu.async_copy(
            bufs[pb], out_hbm.at[pl.ds(base + (g - 1) * G, G)], osems[pb])
    lb = (n_chunks - 1) & 1
    gh[lb].wait()
    oh[lb] = pltpu.async_copy(
        bufs[lb], out_hbm.at[pl.ds(base + (n_chunks - 1) * G, G)], osems[lb])
    oh[(n_chunks - 2) & 1].wait()
    oh[lb].wait()

  return k


def kernel(input, weights):
  bsz, seq_len = input.shape
  dim = weights.shape[1]
  k = _build(bsz, seq_len, dim)
  out = k(input.reshape(-1), weights)
  return out.reshape(bsz, seq_len, dim)
